# baseline (speedup 1.0000x reference)
import jax
import jax.numpy as jnp
import numpy as np
from jax.experimental import pallas as pl
from jax.experimental.pallas import tpu as pltpu

_SIGMA = 4.0
_R = int(np.ceil(3.0 * _SIGMA))
_N = 512
_H = _N // 2


def _blur_matrix():
    x = np.arange(-_R, _R + 1, dtype=np.float64)
    w = np.exp(-0.5 * (x / _SIGMA) ** 2)
    w = w / w.sum()
    b = np.zeros((_N, _N), dtype=np.float64)
    rows = np.arange(_N)
    for t in range(2 * _R + 1):
        cols = np.clip(rows + t - _R, 0, _N - 1)
        np.add.at(b, (rows, cols), w[t])
    return b


_B = _blur_matrix()


def _blur_body(img_hbm, b_hbm, out_hbm, img_v, b_v, tmp_v, out_v,
               sem_b, sem_i0, sem_i1, sem_o0, sem_o1):
    cp_b = pltpu.make_async_copy(b_hbm, b_v, sem_b)
    cp_i0 = pltpu.make_async_copy(img_hbm.at[0, :, 0:_H], img_v.at[:, 0:_H],
                                  sem_i0)
    cp_i1 = pltpu.make_async_copy(img_hbm.at[0, :, _H:_N], img_v.at[:, _H:_N],
                                  sem_i1)
    cp_b.start()
    cp_i0.start()
    cp_i1.start()

    cp_b.wait()
    cp_i0.wait()
    b = b_v[...]
    tmp_v[:, 0:_H] = jax.lax.dot(
        b, img_v[:, 0:_H].astype(jnp.bfloat16),
        preferred_element_type=jnp.float32).astype(jnp.bfloat16)
    cp_i1.wait()
    tmp_v[:, _H:_N] = jax.lax.dot(
        b, img_v[:, _H:_N].astype(jnp.bfloat16),
        preferred_element_type=jnp.float32).astype(jnp.bfloat16)

    out_v[0:_H, :] = jax.lax.dot_general(
        tmp_v[0:_H, :], b, (((1,), (1,)), ((), ())),
        preferred_element_type=jnp.float32)
    cp_o0 = pltpu.make_async_copy(out_v.at[0:_H, :], out_hbm.at[0, 0:_H, :],
                                  sem_o0)
    cp_o0.start()
    out_v[_H:_N, :] = jax.lax.dot_general(
        tmp_v[_H:_N, :], b, (((1,), (1,)), ((), ())),
        preferred_element_type=jnp.float32)
    cp_o1 = pltpu.make_async_copy(out_v.at[_H:_N, :], out_hbm.at[0, _H:_N, :],
                                  sem_o1)
    cp_o1.start()
    cp_o0.wait()
    cp_o1.wait()


@jax.jit
def kernel(img):
    return pl.pallas_call(
        _blur_body,
        in_specs=[
            pl.BlockSpec(memory_space=pltpu.MemorySpace.HBM),
            pl.BlockSpec(memory_space=pltpu.MemorySpace.HBM),
        ],
        out_specs=pl.BlockSpec(memory_space=pltpu.MemorySpace.HBM),
        scratch_shapes=[
            pltpu.VMEM((_N, _N), jnp.float32),
            pltpu.VMEM((_N, _N), jnp.bfloat16),
            pltpu.VMEM((_N, _N), jnp.bfloat16),
            pltpu.VMEM((_N, _N), jnp.float32),
            pltpu.SemaphoreType.DMA,
            pltpu.SemaphoreType.DMA,
            pltpu.SemaphoreType.DMA,
            pltpu.SemaphoreType.DMA,
            pltpu.SemaphoreType.DMA,
        ],
        out_shape=jax.ShapeDtypeStruct((1, _N, _N), jnp.float32),
    )(img, jnp.asarray(_B, dtype=jnp.bfloat16))


# speedup vs baseline: 1.0120x; 1.0120x over previous
import jax
import jax.numpy as jnp
import numpy as np
from jax.experimental import pallas as pl
from jax.experimental.pallas import tpu as pltpu

_SIGMA = 4.0
_R = int(np.ceil(3.0 * _SIGMA))
_N = 512
_H = _N // 2


def _blur_matrix():
    x = np.arange(-_R, _R + 1, dtype=np.float64)
    w = np.exp(-0.5 * (x / _SIGMA) ** 2)
    w = w / w.sum()
    b = np.zeros((_N, _N), dtype=np.float64)
    rows = np.arange(_N)
    for t in range(2 * _R + 1):
        cols = np.clip(rows + t - _R, 0, _N - 1)
        np.add.at(b, (rows, cols), w[t])
    return b


_B = _blur_matrix()


def _blur_body(img_hbm, b_hbm, out_hbm, img_v, b_v, s_v, out_v,
               sem_b, sem_i0, sem_i1, sem_o0, sem_o1):
    cp_b = pltpu.make_async_copy(b_hbm, b_v, sem_b)
    cp_i0 = pltpu.make_async_copy(img_hbm.at[0, 0:_H, :], img_v.at[0:_H, :],
                                  sem_i0)
    cp_i1 = pltpu.make_async_copy(img_hbm.at[0, _H:_N, :], img_v.at[_H:_N, :],
                                  sem_i1)
    cp_b.start()
    cp_i0.start()
    cp_i1.start()

    cp_b.wait()
    b = b_v[...]
    cp_i0.wait()
    s_v[0:_H, :] = jax.lax.dot_general(
        img_v[0:_H, :].astype(jnp.bfloat16), b, (((1,), (1,)), ((), ())),
        preferred_element_type=jnp.float32).astype(jnp.bfloat16)
    cp_i1.wait()
    s_v[_H:_N, :] = jax.lax.dot_general(
        img_v[_H:_N, :].astype(jnp.bfloat16), b, (((1,), (1,)), ((), ())),
        preferred_element_type=jnp.float32).astype(jnp.bfloat16)

    s = s_v[...]
    out_v[0:_H, :] = jax.lax.dot(b[0:_H, :], s,
                                 preferred_element_type=jnp.float32)
    cp_o0 = pltpu.make_async_copy(out_v.at[0:_H, :], out_hbm.at[0, 0:_H, :],
                                  sem_o0)
    cp_o0.start()
    out_v[_H:_N, :] = jax.lax.dot(b[_H:_N, :], s,
                                  preferred_element_type=jnp.float32)
    cp_o1 = pltpu.make_async_copy(out_v.at[_H:_N, :], out_hbm.at[0, _H:_N, :],
                                  sem_o1)
    cp_o1.start()
    cp_o0.wait()
    cp_o1.wait()


@jax.jit
def kernel(img):
    return pl.pallas_call(
        _blur_body,
        in_specs=[
            pl.BlockSpec(memory_space=pltpu.MemorySpace.HBM),
            pl.BlockSpec(memory_space=pltpu.MemorySpace.HBM),
        ],
        out_specs=pl.BlockSpec(memory_space=pltpu.MemorySpace.HBM),
        scratch_shapes=[
            pltpu.VMEM((_N, _N), jnp.float32),
            pltpu.VMEM((_N, _N), jnp.bfloat16),
            pltpu.VMEM((_N, _N), jnp.bfloat16),
            pltpu.VMEM((_N, _N), jnp.float32),
            pltpu.SemaphoreType.DMA,
            pltpu.SemaphoreType.DMA,
            pltpu.SemaphoreType.DMA,
            pltpu.SemaphoreType.DMA,
            pltpu.SemaphoreType.DMA,
        ],
        out_shape=jax.ShapeDtypeStruct((1, _N, _N), jnp.float32),
    )(img, jnp.asarray(_B, dtype=jnp.bfloat16))


# speedup vs baseline: 1.1063x; 1.0933x over previous
import jax
import jax.numpy as jnp
import numpy as np
from jax.experimental import pallas as pl
from jax.experimental.pallas import tpu as pltpu

_SIGMA = 4.0
_R = int(np.ceil(3.0 * _SIGMA))
_N = 512
_BLK = 128
_WIN = 160
_NBLK = _N // _BLK


def _band_starts():
    return [min(max(ib * _BLK - 16, 0), _N - _WIN) for ib in range(_NBLK)]


def _packed_band():
    x = np.arange(-_R, _R + 1, dtype=np.float64)
    w = np.exp(-0.5 * (x / _SIGMA) ** 2)
    w = w / w.sum()
    b = np.zeros((_N, _N), dtype=np.float64)
    rows = np.arange(_N)
    for t in range(2 * _R + 1):
        cols = np.clip(rows + t - _R, 0, _N - 1)
        np.add.at(b, (rows, cols), w[t])
    bp = np.zeros((_NBLK, _BLK, _WIN), dtype=np.float64)
    for ib, st in enumerate(_band_starts()):
        bp[ib] = b[ib * _BLK:(ib + 1) * _BLK, st:st + _WIN]
    return bp


_BP = _packed_band()
_STARTS = _band_starts()


def _blur_body(img_ref, bp_ref, out_ref, s_ref):
    img16 = img_ref[0].astype(jnp.bfloat16)
    for ib, st in enumerate(_STARTS):
        s_ref[ib * _BLK:(ib + 1) * _BLK, :] = jax.lax.dot(
            bp_ref[ib], img16[st:st + _WIN, :],
            preferred_element_type=jnp.float32).astype(jnp.bfloat16)
    s = s_ref[...]
    for jb, st in enumerate(_STARTS):
        out_ref[0, :, jb * _BLK:(jb + 1) * _BLK] = jax.lax.dot_general(
            s[:, st:st + _WIN], bp_ref[jb], (((1,), (1,)), ((), ())),
            preferred_element_type=jnp.float32)


@jax.jit
def kernel(img):
    return pl.pallas_call(
        _blur_body,
        scratch_shapes=[pltpu.VMEM((_N, _N), jnp.bfloat16)],
        out_shape=jax.ShapeDtypeStruct((1, _N, _N), jnp.float32),
    )(img, jnp.asarray(_BP, dtype=jnp.bfloat16))


# speedup vs baseline: 1.1423x; 1.0325x over previous
import jax
import jax.numpy as jnp
import numpy as np
from jax.experimental import pallas as pl
from jax.experimental.pallas import tpu as pltpu

_SIGMA = 4.0
_R = int(np.ceil(3.0 * _SIGMA))
_N = 512
_BLK = 128
_WIN = 160
_NBLK = _N // _BLK


def _band_starts():
    return [min(max(ib * _BLK - 16, 0), _N - _WIN) for ib in range(_NBLK)]


def _packed_band():
    x = np.arange(-_R, _R + 1, dtype=np.float64)
    w = np.exp(-0.5 * (x / _SIGMA) ** 2)
    w = w / w.sum()
    b = np.zeros((_N, _N), dtype=np.float64)
    rows = np.arange(_N)
    for t in range(2 * _R + 1):
        cols = np.clip(rows + t - _R, 0, _N - 1)
        np.add.at(b, (rows, cols), w[t])
    bp = np.zeros((_NBLK, _BLK, _WIN), dtype=np.float64)
    for ib, st in enumerate(_band_starts()):
        bp[ib] = b[ib * _BLK:(ib + 1) * _BLK, st:st + _WIN]
    return bp


_BP = _packed_band()
_STARTS = _band_starts()


def _blur_body(img_ref, bp_ref, out_hbm, s_ref, o_ref,
               sem0, sem1, sem2, sem3):
    sems = (sem0, sem1, sem2, sem3)
    img16 = img_ref[0].astype(jnp.bfloat16)
    for ib, st in enumerate(_STARTS):
        s_ref[ib * _BLK:(ib + 1) * _BLK, :] = jax.lax.dot(
            bp_ref[ib], img16[st:st + _WIN, :],
            preferred_element_type=jnp.float32).astype(jnp.bfloat16)
    s = s_ref[...]
    copies = []
    for jb, st in enumerate(_STARTS):
        sl = slice(jb * _BLK, (jb + 1) * _BLK)
        o_ref[:, sl] = jax.lax.dot_general(
            s[:, st:st + _WIN], bp_ref[jb], (((1,), (1,)), ((), ())),
            preferred_element_type=jnp.float32)
        cp = pltpu.make_async_copy(o_ref.at[:, sl], out_hbm.at[0, :, sl],
                                   sems[jb])
        cp.start()
        copies.append(cp)
    for cp in copies:
        cp.wait()


@jax.jit
def kernel(img):
    return pl.pallas_call(
        _blur_body,
        out_specs=pl.BlockSpec(memory_space=pltpu.MemorySpace.HBM),
        scratch_shapes=[
            pltpu.VMEM((_N, _N), jnp.bfloat16),
            pltpu.VMEM((_N, _N), jnp.float32),
            pltpu.SemaphoreType.DMA,
            pltpu.SemaphoreType.DMA,
            pltpu.SemaphoreType.DMA,
            pltpu.SemaphoreType.DMA,
        ],
        out_shape=jax.ShapeDtypeStruct((1, _N, _N), jnp.float32),
    )(img, jnp.asarray(_BP, dtype=jnp.bfloat16))
